# baseline (speedup 1.0000x reference)
import jax
import jax.numpy as jnp
from jax import lax
from jax.experimental import pallas as pl
from jax.experimental.pallas import tpu as pltpu
from jax.experimental.pallas import tpu_sc as plsc

L = 200
N = 1024
D = 64
R = L * N
NUM_CORES = 2
NUM_SUBCORES = 16
NW = NUM_CORES * NUM_SUBCORES
ROWS_PER_W = R // NW
CHUNK = 128
CHUNKS = ROWS_PER_W // CHUNK
PAIRS = CHUNKS // 2
LANES = 16
KG = D // LANES


def _tc_transpose_body(x_ref, s_ref, xt_ref, st_ref):
    xt_ref[...] = jnp.transpose(x_ref[...], (1, 0))[None]
    st_ref[...] = jnp.transpose(s_ref[...], (1, 0))[None]


def _tc_retile_body(in_ref, out_ref):
    blk = in_ref[...]
    a = blk[:, None, :D]
    b = blk[:, None, D:]
    pair = jnp.concatenate([a, b], axis=1)
    out_ref[...] = pair.reshape(1, N, D)


def _sc_body(xt_hbm, st_hbm, tok_hbm, seg_hbm, pos_hbm, out_hbm,
             idx_all, seg_all, pos_v, segt_b, d_b,
             tok0, tok1, ob0, ob1, c00, c01,
             semg0, semg1, semo0, semo1):
    toks = (tok0, tok1)
    obs = (ob0, ob1)
    c0s = (c00, c01)
    semgs = (semg0, semg1)
    semos = (semo0, semo1)

    wid = lax.axis_index("s") * NUM_CORES + lax.axis_index("c")
    a_blk = wid // 4
    l_blk = wid % 4
    wrow = a_blk * L + l_blk * CHUNKS
    pltpu.sync_copy(xt_hbm.at[pl.ds(wrow, CHUNKS)], idx_all)
    pltpu.sync_copy(st_hbm.at[pl.ds(wrow, CHUNKS)], seg_all)
    pltpu.sync_copy(pos_hbm.at[pl.ds(0, L)], pos_v)
    pltpu.sync_copy(seg_hbm, segt_b)
    for k in range(KG):
        ksl = pl.ds(k * LANES, LANES)
        d_b[0, ksl] = segt_b[1, ksl] - segt_b[0, ksl]

    def issue(g, b):
        pltpu.async_copy(tok_hbm.at[idx_all.at[g]], toks[b], semgs[b])

    def wait_gather(b):
        pltpu.make_async_copy(tok_hbm.at[idx_all.at[0]], toks[b],
                              semgs[b]).wait()

    def wait_out(b):
        pltpu.make_async_copy(obs[b], out_hbm.at[pl.ds(0, CHUNK // 2)],
                              semos[b]).wait()

    def compute(g, b):
        tok = toks[b]
        ob = obs[b]
        c0 = c0s[b]
        l = l_blk * CHUNKS + g
        for k in range(KG):
            ksl = pl.ds(k * LANES, LANES)
            c0[0, ksl] = pos_v[l, ksl] + segt_b[0, ksl]

        def grp_body(gi, carry):
            rbase = gi * LANES
            svf = seg_all[g, pl.ds(rbase, LANES)].astype(jnp.float32)
            for j in range(LANES):
                spl = jnp.full((LANES,), svf[j], dtype=jnp.float32)
                r = rbase + j
                ocol = (j % 2) * D
                for k in range(KG):
                    ksl = pl.ds(k * LANES, LANES)
                    ob[gi * (LANES // 2) + j // 2,
                       pl.ds(ocol + k * LANES, LANES)] = (
                        tok[r, ksl] + c0[0, ksl] + spl * d_b[0, ksl])
            return carry

        lax.fori_loop(0, CHUNK // LANES, grp_body, 0)

    issue(0, 0)
    issue(1, 1)

    def pair_body(go, carry):
        for b in (0, 1):
            g = 2 * go + b
            wait_gather(b)

            @pl.when(go > 0)
            def _():
                wait_out(b)

            compute(g, b)
            l = l_blk * CHUNKS + g
            pltpu.async_copy(
                obs[b],
                out_hbm.at[pl.ds(l * (N // 2) + a_blk * (CHUNK // 2),
                                 CHUNK // 2)],
                semos[b])

            @pl.when(go < PAIRS - 1)
            def _():
                issue(g + 2, b)

        return carry

    lax.fori_loop(0, PAIRS, pair_body, 0)
    wait_out(0)
    wait_out(1)


def kernel(x, segments, token_table, segment_table, pos_embedding):
    pos = pos_embedding[:, 0, :]
    xt3, st3 = pl.pallas_call(
        _tc_transpose_body,
        grid=(N // 128,),
        in_specs=[pl.BlockSpec((128, L), lambda a: (a, 0))] * 2,
        out_specs=[pl.BlockSpec((1, L, 128), lambda a: (a, 0, 0))] * 2,
        out_shape=[jax.ShapeDtypeStruct((N // 128, L, 128), jnp.int32)] * 2,
    )(x.astype(jnp.int32), segments.astype(jnp.int32))
    xt2 = xt3.reshape(R // 128, 128)
    st2 = st3.reshape(R // 128, 128)

    mesh = plsc.VectorSubcoreMesh(core_axis_name="c", subcore_axis_name="s")
    flat = pl.kernel(
        _sc_body,
        out_type=jax.ShapeDtypeStruct((R // 2, 2 * D), jnp.float32),
        mesh=mesh,
        scratch_types=[
            pltpu.VMEM((CHUNKS, CHUNK), jnp.int32),
            pltpu.VMEM((CHUNKS, CHUNK), jnp.int32),
            pltpu.VMEM((L, D), jnp.float32),
            pltpu.VMEM((2, D), jnp.float32),
            pltpu.VMEM((1, D), jnp.float32),
            pltpu.VMEM((CHUNK, D), jnp.float32),
            pltpu.VMEM((CHUNK, D), jnp.float32),
            pltpu.VMEM((CHUNK // 2, 2 * D), jnp.float32),
            pltpu.VMEM((CHUNK // 2, 2 * D), jnp.float32),
            pltpu.VMEM((1, D), jnp.float32),
            pltpu.VMEM((1, D), jnp.float32),
            pltpu.SemaphoreType.DMA,
            pltpu.SemaphoreType.DMA,
            pltpu.SemaphoreType.DMA,
            pltpu.SemaphoreType.DMA,
        ],
        compiler_params=pltpu.CompilerParams(use_tc_tiling_on_sc=False),
    )(xt2, st2, token_table, segment_table, pos)

    out = pl.pallas_call(
        _tc_retile_body,
        grid=(L,),
        in_specs=[pl.BlockSpec((N * D // 128, 128), lambda i: (i, 0))],
        out_specs=pl.BlockSpec((1, N, D), lambda i: (i, 0, 0)),
        out_shape=jax.ShapeDtypeStruct((L, N, D), jnp.float32),
    )(flat)
    return out


# speedup vs baseline: 1.1473x; 1.1473x over previous
import jax
import jax.numpy as jnp
from jax import lax
from jax.experimental import pallas as pl
from jax.experimental.pallas import tpu as pltpu
from jax.experimental.pallas import tpu_sc as plsc

L = 200
N = 1024
D = 64
R = L * N
NUM_CORES = 2
NUM_SUBCORES = 16
NW = NUM_CORES * NUM_SUBCORES
ROWS_PER_W = R // NW
CHUNK = 128
CHUNKS = ROWS_PER_W // CHUNK
PAIRS = CHUNKS // 2
LANES = 16
KG = D // LANES


def _tc_transpose_body(x_ref, s_ref, xt_ref, st_ref):
    xt_ref[...] = jnp.transpose(x_ref[...], (1, 0))[None]
    st_ref[...] = jnp.transpose(s_ref[...], (1, 0))[None]


def _sc_body(xt_hbm, st_hbm, tok_hbm, seg_hbm, pos_hbm, out_hbm,
             idx_all, seg_all, pos_v, segt_b, d_b,
             tok0, tok1, ob0, ob1, c00, c01,
             semg0, semg1, semo0, semo1):
    toks = (tok0, tok1)
    obs = (ob0, ob1)
    c0s = (c00, c01)
    semgs = (semg0, semg1)
    semos = (semo0, semo1)

    wid = lax.axis_index("s") * NUM_CORES + lax.axis_index("c")
    a_blk = wid // 4
    l_blk = wid % 4
    wflat = (a_blk * L + l_blk * CHUNKS) * CHUNK
    pltpu.sync_copy(xt_hbm.at[pl.ds(wflat, ROWS_PER_W)], idx_all)
    pltpu.sync_copy(st_hbm.at[pl.ds(wflat, ROWS_PER_W)], seg_all)
    pltpu.sync_copy(pos_hbm.at[pl.ds(0, L)], pos_v)
    pltpu.sync_copy(seg_hbm, segt_b)
    for k in range(KG):
        ksl = pl.ds(k * LANES, LANES)
        d_b[0, ksl] = segt_b[1, ksl] - segt_b[0, ksl]

    def issue(g, b):
        pltpu.async_copy(tok_hbm.at[idx_all.at[pl.ds(g * CHUNK, CHUNK)]],
                         toks[b], semgs[b])

    def wait_gather(b):
        pltpu.make_async_copy(tok_hbm.at[idx_all.at[pl.ds(0, CHUNK)]],
                              toks[b], semgs[b]).wait()

    def wait_out(b):
        pltpu.make_async_copy(obs[b], out_hbm.at[pl.ds(0, CHUNK)],
                              semos[b]).wait()

    def compute(g, b):
        tok = toks[b]
        ob = obs[b]
        c0 = c0s[b]
        l = l_blk * CHUNKS + g
        for k in range(KG):
            ksl = pl.ds(k * LANES, LANES)
            c0[0, ksl] = pos_v[l, ksl] + segt_b[0, ksl]

        def grp_body(gi, carry):
            rbase = gi * LANES
            svf = seg_all[pl.ds(g * CHUNK + rbase, LANES)].astype(jnp.float32)
            for j in range(LANES):
                spl = jnp.full((LANES,), svf[j], dtype=jnp.float32)
                r = rbase + j
                for k in range(KG):
                    ksl = pl.ds(k * LANES, LANES)
                    ob[r, ksl] = tok[r, ksl] + c0[0, ksl] + spl * d_b[0, ksl]
            return carry

        lax.fori_loop(0, CHUNK // LANES, grp_body, 0)

    issue(0, 0)
    issue(1, 1)

    def pair_body(go, carry):
        for b in (0, 1):
            g = 2 * go + b
            wait_gather(b)

            @pl.when(go > 0)
            def _():
                wait_out(b)

            compute(g, b)
            l = l_blk * CHUNKS + g
            pltpu.async_copy(
                obs[b],
                out_hbm.at[pl.ds(l * N + a_blk * CHUNK, CHUNK)],
                semos[b])

            @pl.when(go < PAIRS - 1)
            def _():
                issue(g + 2, b)

        return carry

    lax.fori_loop(0, PAIRS, pair_body, 0)
    wait_out(0)
    wait_out(1)


def kernel(x, segments, token_table, segment_table, pos_embedding):
    pos = pos_embedding[:, 0, :]
    xt3, st3 = pl.pallas_call(
        _tc_transpose_body,
        grid=(N // 128,),
        in_specs=[pl.BlockSpec((128, L), lambda a: (a, 0))] * 2,
        out_specs=[pl.BlockSpec((1, L, 128), lambda a: (a, 0, 0))] * 2,
        out_shape=[jax.ShapeDtypeStruct((N // 128, L, 128), jnp.int32)] * 2,
    )(x.astype(jnp.int32), segments.astype(jnp.int32))
    xt = xt3.reshape(R)
    st = st3.reshape(R)

    mesh = plsc.VectorSubcoreMesh(core_axis_name="c", subcore_axis_name="s")
    flat = pl.kernel(
        _sc_body,
        out_type=jax.ShapeDtypeStruct((R, D), jnp.float32),
        mesh=mesh,
        scratch_types=[
            pltpu.VMEM((ROWS_PER_W,), jnp.int32),
            pltpu.VMEM((ROWS_PER_W,), jnp.int32),
            pltpu.VMEM((L, D), jnp.float32),
            pltpu.VMEM((2, D), jnp.float32),
            pltpu.VMEM((1, D), jnp.float32),
            pltpu.VMEM((CHUNK, D), jnp.float32),
            pltpu.VMEM((CHUNK, D), jnp.float32),
            pltpu.VMEM((CHUNK, D), jnp.float32),
            pltpu.VMEM((CHUNK, D), jnp.float32),
            pltpu.VMEM((1, D), jnp.float32),
            pltpu.VMEM((1, D), jnp.float32),
            pltpu.SemaphoreType.DMA,
            pltpu.SemaphoreType.DMA,
            pltpu.SemaphoreType.DMA,
            pltpu.SemaphoreType.DMA,
        ],
        compiler_params=pltpu.CompilerParams(use_tc_tiling_on_sc=False),
    )(xt, st, token_table, segment_table, pos)
    return flat.reshape(L, N, D)
